# baseline (speedup 1.0000x reference)
import functools

import jax
import jax.numpy as jnp
import numpy as np
from jax import lax
from jax.experimental import pallas as pl
from jax.experimental.pallas import tpu as pltpu
from jax.experimental.pallas import tpu_sc as plsc

N_IMG = 256
NC = 8
NB = 4
NK = 262144
CHUNK = 8192
UNROLL = 8


def _dft_mats():
    k = np.arange(N_IMG) - N_IMG // 2
    m = np.outer(k, k).astype(np.float64)
    f = np.exp(-2j * np.pi * m / N_IMG) / np.sqrt(N_IMG)
    return (jnp.asarray(f.real, dtype=jnp.float32),
            jnp.asarray(f.imag, dtype=jnp.float32))


def _fft_body(fr_ref, fi_ref, x_ref, out_ref):
    x = x_ref[0]
    fr = fr_ref[...]
    fi = fi_ref[...]
    ar = jnp.dot(fr, x, preferred_element_type=jnp.float32)
    ai = jnp.dot(fi, x, preferred_element_type=jnp.float32)
    out_ref[0, 0] = (jnp.dot(ar, fr, preferred_element_type=jnp.float32)
                     - jnp.dot(ai, fi, preferred_element_type=jnp.float32))
    out_ref[1, 0] = (jnp.dot(ar, fi, preferred_element_type=jnp.float32)
                     + jnp.dot(ai, fr, preferred_element_type=jnp.float32))


def _centered_fft2(img_flat):
    fr, fi = _dft_mats()
    nimg = img_flat.shape[0]
    return pl.pallas_call(
        _fft_body,
        grid=(nimg,),
        in_specs=[
            pl.BlockSpec((N_IMG, N_IMG), lambda b: (0, 0)),
            pl.BlockSpec((N_IMG, N_IMG), lambda b: (0, 0)),
            pl.BlockSpec((1, N_IMG, N_IMG), lambda b: (b, 0, 0)),
        ],
        out_specs=pl.BlockSpec((2, 1, N_IMG, N_IMG), lambda b: (0, b, 0, 0)),
        out_shape=jax.ShapeDtypeStruct((2, nimg, N_IMG, N_IMG), jnp.float32),
    )(fr, fi, img_flat)


def _sc_gather(grids, idx):
    mesh = plsc.VectorSubcoreMesh(core_axis_name="c", subcore_axis_name="s")

    nchunks = NK // CHUNK

    @functools.partial(
        pl.kernel,
        mesh=mesh,
        out_type=(jax.ShapeDtypeStruct((NB * NC, NK), jnp.float32),
                  jax.ShapeDtypeStruct((NB * NC, NK), jnp.float32)),
        scratch_types=[
            pltpu.VMEM((N_IMG * N_IMG,), jnp.float32),
            pltpu.VMEM((CHUNK // 2,), jnp.int32),
            pltpu.VMEM((CHUNK // 2,), jnp.int32),
            pltpu.VMEM((2, CHUNK), jnp.float32),
            pltpu.SemaphoreType.DMA,
            pltpu.SemaphoreType.DMA,
            pltpu.SemaphoreType.DMA,
            pltpu.SemaphoreType.DMA,
        ],
        compiler_params=pltpu.CompilerParams(needs_layout_passes=False),
    )
    def gather_kernel(grids_hbm, idx_hbm, re_hbm, im_hbm,
                      grid_v, idx0_v, idx1_v, out_v,
                      isem0, isem1, osem0, osem1):
        wid = lax.axis_index("s") * 2 + lax.axis_index("c")
        idx_vs = (idx0_v, idx1_v)
        isems = (isem0, isem1)
        osems = (osem0, osem1)
        q = wid
        n = q // NC
        for isim in range(2):
            pltpu.sync_copy(grids_hbm.at[isim * 32 + q], grid_v)
            dst = re_hbm if isim == 0 else im_hbm

            hchunk = CHUNK // 2
            idx_h = pltpu.async_copy(
                idx_hbm.at[pl.ds(n * (NK // 2), hchunk)], idx_vs[0], isems[0])
            out_hs = [None, None]
            for kc in range(nchunks):
                b = kc % 2
                nxt = None
                if kc + 1 < nchunks:
                    nxt = pltpu.async_copy(
                        idx_hbm.at[pl.ds(n * (NK // 2) + (kc + 1) * hchunk,
                                         hchunk)],
                        idx_vs[1 - b], isems[1 - b])
                idx_h.wait()
                if out_hs[b] is not None:
                    out_hs[b].wait()

                @plsc.parallel_loop(0, hchunk, 16, unroll=UNROLL)
                def g_body(po, b=b):
                    w = idx_vs[b][pl.ds(po, 16)]
                    lo = w & 0xFFFF
                    hi = lax.shift_right_logical(w, 16)
                    out_v[b, pl.ds(2 * po, 16)] = plsc.load_gather(
                        grid_v, [lo])
                    out_v[b, pl.ds(2 * po + 16, 16)] = plsc.load_gather(
                        grid_v, [hi])
                out_hs[b] = pltpu.async_copy(
                    out_v.at[b], dst.at[q, pl.ds(kc * CHUNK, CHUNK)], osems[b])
                idx_h = nxt
            for h in out_hs:
                if h is not None:
                    h.wait()

    return gather_kernel(grids, idx)


def kernel(img, trj):
    nb, nc = img.shape[0], img.shape[1]
    img_flat = img.reshape(nb * nc, N_IMG, N_IMG)
    grids = _centered_fft2(img_flat)
    grids = grids.reshape(2 * nb * nc, N_IMG * N_IMG)
    idx = trj[..., 0] * N_IMG + trj[..., 1]
    g = idx.reshape(nb, NK // 32, 2, 16)
    idx = (g[:, :, 0, :] | (g[:, :, 1, :] << 16)).reshape(nb * (NK // 2))
    re, im = _sc_gather(grids, idx)
    return lax.complex(re, im).reshape(nb, nc, NK)


# speedup vs baseline: 1.3485x; 1.3485x over previous
import functools

import jax
import jax.numpy as jnp
import numpy as np
from jax import lax
from jax.experimental import pallas as pl
from jax.experimental.pallas import tpu as pltpu
from jax.experimental.pallas import tpu_sc as plsc

N_IMG = 256
NC = 8
NB = 4
NK = 262144
CHUNK = 8192
UNROLL = 8


def _dft_mats():
    k = np.arange(N_IMG) - N_IMG // 2
    m = np.outer(k, k).astype(np.float64)
    f = np.exp(-2j * np.pi * m / N_IMG) / np.sqrt(N_IMG)
    return (jnp.asarray(f.real, dtype=jnp.float32),
            jnp.asarray(f.imag, dtype=jnp.float32))


def _fft_body(fr_ref, fi_ref, x_ref, out_ref):
    x = x_ref[0]
    fr = fr_ref[...]
    fi = fi_ref[...]
    ar = jnp.dot(fr, x, preferred_element_type=jnp.float32)
    ai = jnp.dot(fi, x, preferred_element_type=jnp.float32)
    out_ref[0, 0] = (jnp.dot(ar, fr, preferred_element_type=jnp.float32)
                     - jnp.dot(ai, fi, preferred_element_type=jnp.float32))
    out_ref[1, 0] = (jnp.dot(ar, fi, preferred_element_type=jnp.float32)
                     + jnp.dot(ai, fr, preferred_element_type=jnp.float32))


def _centered_fft2(img_flat):
    fr, fi = _dft_mats()
    nimg = img_flat.shape[0]
    return pl.pallas_call(
        _fft_body,
        grid=(nimg,),
        in_specs=[
            pl.BlockSpec((N_IMG, N_IMG), lambda b: (0, 0)),
            pl.BlockSpec((N_IMG, N_IMG), lambda b: (0, 0)),
            pl.BlockSpec((1, N_IMG, N_IMG), lambda b: (b, 0, 0)),
        ],
        out_specs=pl.BlockSpec((2, 1, N_IMG, N_IMG), lambda b: (0, b, 0, 0)),
        out_shape=jax.ShapeDtypeStruct((2, nimg, N_IMG, N_IMG), jnp.float32),
    )(fr, fi, img_flat)


def _sc_gather(grids, idx):
    mesh = plsc.VectorSubcoreMesh(core_axis_name="c", subcore_axis_name="s")

    nchunks = NK // CHUNK

    @functools.partial(
        pl.kernel,
        mesh=mesh,
        out_type=(jax.ShapeDtypeStruct((NB * NC, NK), jnp.float32),
                  jax.ShapeDtypeStruct((NB * NC, NK), jnp.float32)),
        scratch_types=[
            pltpu.VMEM((N_IMG * N_IMG,), jnp.float32),
            pltpu.VMEM((CHUNK,), jnp.int32),
            pltpu.VMEM((CHUNK,), jnp.int32),
            pltpu.VMEM((2, CHUNK), jnp.float32),
            pltpu.SemaphoreType.DMA,
            pltpu.SemaphoreType.DMA,
            pltpu.SemaphoreType.DMA,
            pltpu.SemaphoreType.DMA,
        ],
        compiler_params=pltpu.CompilerParams(needs_layout_passes=False),
    )
    def gather_kernel(grids_hbm, idx_hbm, re_hbm, im_hbm,
                      grid_v, idx0_v, idx1_v, out_v,
                      isem0, isem1, osem0, osem1):
        wid = lax.axis_index("s") * 2 + lax.axis_index("c")
        idx_vs = (idx0_v, idx1_v)
        isems = (isem0, isem1)
        osems = (osem0, osem1)
        q = wid
        n = q // NC
        for isim in range(2):
            pltpu.sync_copy(grids_hbm.at[isim * 32 + q], grid_v)
            dst = re_hbm if isim == 0 else im_hbm

            idx_h = pltpu.async_copy(
                idx_hbm.at[pl.ds(n * NK, CHUNK)], idx_vs[0], isems[0])
            out_hs = [None, None]
            for kc in range(nchunks):
                b = kc % 2
                nxt = None
                if kc + 1 < nchunks:
                    nxt = pltpu.async_copy(
                        idx_hbm.at[pl.ds(n * NK + (kc + 1) * CHUNK, CHUNK)],
                        idx_vs[1 - b], isems[1 - b])
                idx_h.wait()
                if out_hs[b] is not None:
                    out_hs[b].wait()

                @plsc.parallel_loop(0, CHUNK, 16, unroll=UNROLL)
                def g_body(off, b=b):
                    iv = idx_vs[b][pl.ds(off, 16)]
                    out_v[b, pl.ds(off, 16)] = plsc.load_gather(grid_v, [iv])
                out_hs[b] = pltpu.async_copy(
                    out_v.at[b], dst.at[q, pl.ds(kc * CHUNK, CHUNK)], osems[b])
                idx_h = nxt
            for h in out_hs:
                if h is not None:
                    h.wait()

    return gather_kernel(grids, idx)


def kernel(img, trj):
    nb, nc = img.shape[0], img.shape[1]
    img_flat = img.reshape(nb * nc, N_IMG, N_IMG)
    grids = _centered_fft2(img_flat)
    grids = grids.reshape(2 * nb * nc, N_IMG * N_IMG)
    idx = (trj[..., 0] * N_IMG + trj[..., 1]).reshape(nb * NK)
    re, im = _sc_gather(grids, idx)
    return lax.complex(re, im).reshape(nb, nc, NK)


# speedup vs baseline: 1.3661x; 1.0130x over previous
import functools

import jax
import jax.numpy as jnp
import numpy as np
from jax import lax
from jax.experimental import pallas as pl
from jax.experimental.pallas import tpu as pltpu
from jax.experimental.pallas import tpu_sc as plsc

N_IMG = 256
NC = 8
NB = 4
NK = 262144
CHUNK = 8192
UNROLL = 8


def _dft_mats():
    k = np.arange(N_IMG) - N_IMG // 2
    m = np.outer(k, k).astype(np.float64)
    f = np.exp(-2j * np.pi * m / N_IMG) / np.sqrt(N_IMG)
    return (jnp.asarray(f.real, dtype=jnp.float32),
            jnp.asarray(f.imag, dtype=jnp.float32))


def _fft_body(fr_ref, fi_ref, x_ref, out_ref):
    x = x_ref[0]
    fr = fr_ref[...]
    fi = fi_ref[...]
    ar = jnp.dot(fr, x, preferred_element_type=jnp.float32)
    ai = jnp.dot(fi, x, preferred_element_type=jnp.float32)
    out_ref[0, 0] = (jnp.dot(ar, fr, preferred_element_type=jnp.float32)
                     - jnp.dot(ai, fi, preferred_element_type=jnp.float32))
    out_ref[1, 0] = (jnp.dot(ar, fi, preferred_element_type=jnp.float32)
                     + jnp.dot(ai, fr, preferred_element_type=jnp.float32))


def _centered_fft2(img_flat):
    fr, fi = _dft_mats()
    nimg = img_flat.shape[0]
    return pl.pallas_call(
        _fft_body,
        grid=(nimg,),
        in_specs=[
            pl.BlockSpec((N_IMG, N_IMG), lambda b: (0, 0)),
            pl.BlockSpec((N_IMG, N_IMG), lambda b: (0, 0)),
            pl.BlockSpec((1, N_IMG, N_IMG), lambda b: (b, 0, 0)),
        ],
        out_specs=pl.BlockSpec((2, 1, N_IMG, N_IMG), lambda b: (0, b, 0, 0)),
        out_shape=jax.ShapeDtypeStruct((2, nimg, N_IMG, N_IMG), jnp.float32),
    )(fr, fi, img_flat)


def _sc_gather(grids, idx):
    mesh = plsc.VectorSubcoreMesh(core_axis_name="c", subcore_axis_name="s")

    nchunks = NK // CHUNK

    @functools.partial(
        pl.kernel,
        mesh=mesh,
        out_type=(jax.ShapeDtypeStruct((NB * NC, NK), jnp.float32),
                  jax.ShapeDtypeStruct((NB * NC, NK), jnp.float32)),
        scratch_types=[
            pltpu.VMEM((N_IMG * N_IMG,), jnp.float32),
            pltpu.VMEM((CHUNK,), jnp.int32),
            pltpu.VMEM((CHUNK,), jnp.int32),
            pltpu.VMEM((2, CHUNK), jnp.float32),
            pltpu.VMEM_SHARED((2 * NK,), jnp.int32),
            pltpu.SemaphoreType.DMA,
            pltpu.SemaphoreType.DMA,
            pltpu.SemaphoreType.DMA,
            pltpu.SemaphoreType.DMA,
        ],
        compiler_params=pltpu.CompilerParams(needs_layout_passes=False),
    )
    def gather_kernel(grids_hbm, idx_hbm, re_hbm, im_hbm,
                      grid_v, idx0_v, idx1_v, out_v, idx_sh,
                      isem0, isem1, osem0, osem1):
        cc = lax.axis_index("c")
        s = lax.axis_index("s")
        idx_vs = (idx0_v, idx1_v)
        isems = (isem0, isem1)
        osems = (osem0, osem1)
        q = cc * 16 + s
        n = q // NC
        n_loc = s // NC
        seg = 2 * NK // 16
        pltpu.sync_copy(idx_hbm.at[pl.ds(cc * 2 * NK + s * seg, seg)],
                        idx_sh.at[pl.ds(s * seg, seg)])
        plsc.subcore_barrier()
        for isim in range(2):
            pltpu.sync_copy(grids_hbm.at[isim * 32 + q], grid_v)
            dst = re_hbm if isim == 0 else im_hbm

            idx_h = pltpu.async_copy(
                idx_sh.at[pl.ds(n_loc * NK, CHUNK)], idx_vs[0], isems[0])
            out_hs = [None, None]
            for kc in range(nchunks):
                b = kc % 2
                nxt = None
                if kc + 1 < nchunks:
                    nxt = pltpu.async_copy(
                        idx_sh.at[pl.ds(n_loc * NK + (kc + 1) * CHUNK, CHUNK)],
                        idx_vs[1 - b], isems[1 - b])
                idx_h.wait()
                if out_hs[b] is not None:
                    out_hs[b].wait()

                @plsc.parallel_loop(0, CHUNK, 16, unroll=UNROLL)
                def g_body(off, b=b):
                    iv = idx_vs[b][pl.ds(off, 16)]
                    out_v[b, pl.ds(off, 16)] = plsc.load_gather(grid_v, [iv])

                out_hs[b] = pltpu.async_copy(
                    out_v.at[b], dst.at[q, pl.ds(kc * CHUNK, CHUNK)], osems[b])
                idx_h = nxt
            for h in out_hs:
                if h is not None:
                    h.wait()

    return gather_kernel(grids, idx)


def kernel(img, trj):
    nb, nc = img.shape[0], img.shape[1]
    img_flat = img.reshape(nb * nc, N_IMG, N_IMG)
    grids = _centered_fft2(img_flat)
    grids = grids.reshape(2 * nb * nc, N_IMG * N_IMG)
    idx = (trj[..., 0] * N_IMG + trj[..., 1]).reshape(nb * NK)
    re, im = _sc_gather(grids, idx)
    return lax.complex(re, im).reshape(nb, nc, NK)
